# baseline (speedup 1.0000x reference)
import jax
import jax.numpy as jnp
from jax import lax
from jax.experimental import pallas as pl
from jax.experimental.pallas import tpu as pltpu
from jax.experimental.pallas import tpu_sc as plsc

N_FIELDS = 26
BATCH = 16384
VOCAB = 100000
EMBED_DIM = 128

_NC = 2
_NS = 16
_NW = _NC * _NS
_ROWS_TOTAL = N_FIELDS * BATCH
_RPW = _ROWS_TOTAL // _NW
_RC = 128
_NCHUNK = _RPW // _RC
_NBUF = 4


def _body(tab_hbm, ids_hbm, out_hbm, ids_v, f0, f1, f2, f3, r0, r1, r2, r3,
          gsem, osem):
    c = lax.axis_index("c")
    s = lax.axis_index("s")
    wid = s * _NC + c
    wbase = wid * _RPW

    pltpu.sync_copy(ids_hbm.at[pl.ds(wbase, _RPW)], ids_v)

    fused = (f0, f1, f2, f3)
    rows = (r0, r1, r2, r3)

    def fire_gather(ci, b):
        off = ((wbase + ci * _RC) >> 14) * VOCAB
        for v in range(_RC // 16):
            fused[b][pl.ds(v * 16, 16)] = ids_v[pl.ds(ci * _RC + v * 16, 16)] + off
        pltpu.async_copy(tab_hbm.at[fused[b]], rows[b], gsem)

    def wait_gather(b):
        pltpu.make_async_copy(tab_hbm.at[fused[b]], rows[b], gsem).wait()

    def fire_out(ci, b):
        pltpu.async_copy(rows[b], out_hbm.at[pl.ds(wbase + ci * _RC, _RC)], osem)

    def drain_out(b):
        pltpu.make_async_copy(out_hbm.at[pl.ds(0, _RC)], rows[b], osem).wait()

    fire_gather(0, 0)
    fire_gather(1, 1)
    fire_gather(2, 2)
    wait_gather(0)
    fire_out(0, 0)
    fire_gather(3, 3)

    def step(i, carry):
        ci = _NBUF * i + 1
        for j in range(_NBUF):
            b = (1 + j) % _NBUF
            wait_gather(b)
            fire_out(ci + j, b)
            drain_out((b + 3) % _NBUF)
            fire_gather(ci + j + 3, (b + 3) % _NBUF)
        return carry

    lax.fori_loop(0, (_NCHUNK - _NBUF) // _NBUF, step, 0)

    for j in range(3):
        b = (1 + j) % _NBUF
        wait_gather(b)
        fire_out(_NCHUNK - 3 + j, b)
    for b in range(_NBUF):
        drain_out(b)


@jax.jit
def _lookup(tab_flat, ids_flat):
    run = pl.kernel(
        _body,
        out_type=jax.ShapeDtypeStruct((_ROWS_TOTAL, EMBED_DIM), jnp.float32),
        mesh=plsc.VectorSubcoreMesh(core_axis_name="c", subcore_axis_name="s"),
        compiler_params=pltpu.CompilerParams(needs_layout_passes=False),
        scratch_types=[
            pltpu.VMEM((_RPW,), jnp.int32),
            pltpu.VMEM((_RC,), jnp.int32),
            pltpu.VMEM((_RC,), jnp.int32),
            pltpu.VMEM((_RC,), jnp.int32),
            pltpu.VMEM((_RC,), jnp.int32),
            pltpu.VMEM((_RC, EMBED_DIM), jnp.float32),
            pltpu.VMEM((_RC, EMBED_DIM), jnp.float32),
            pltpu.VMEM((_RC, EMBED_DIM), jnp.float32),
            pltpu.VMEM((_RC, EMBED_DIM), jnp.float32),
            pltpu.SemaphoreType.DMA,
            pltpu.SemaphoreType.DMA,
        ],
    )
    return run(tab_flat, ids_flat)


def kernel(ids, tables):
    ids_flat = ids.astype(jnp.int32).reshape(_ROWS_TOTAL)
    tab_flat = tables.reshape(N_FIELDS * VOCAB, EMBED_DIM)
    out = _lookup(tab_flat, ids_flat)
    return jnp.transpose(out.reshape(N_FIELDS, BATCH, EMBED_DIM), (1, 0, 2))
